# baseline (speedup 1.0000x reference)
import functools

import jax
import jax.numpy as jnp
import numpy as np
from jax import lax
from jax.experimental import pallas as pl
from jax.experimental.pallas import tpu as pltpu
from jax.experimental.pallas import tpu_sc as plsc

_G = 128
_EPS = 1e-5

_NC = 2
_NS = 16
_CH = 56
_LANES = 16


def _pre_body(x_ref, w_ref, b_ref, o_ref):
    o_ref[...] = (
        jnp.dot(x_ref[...], w_ref[...], preferred_element_type=jnp.float32)
        + b_ref[...]
    )


def _pre(x, w, b, blk):
    n, d = x.shape
    return pl.pallas_call(
        _pre_body,
        grid=(n // blk,),
        in_specs=[
            pl.BlockSpec((blk, d), lambda i: (i, 0)),
            pl.BlockSpec((d, d), lambda i: (0, 0)),
            pl.BlockSpec((1, d), lambda i: (0, 0)),
        ],
        out_specs=pl.BlockSpec((blk, d), lambda i: (i, 0)),
        out_shape=jax.ShapeDtypeStruct((n, d), jnp.float32),
    )(x, w, b.reshape(1, d))


def _q_body(ea_ref, w_ref, q_ref):
    q_ref[...] = jnp.dot(
        ea_ref[...], w_ref[...], preferred_element_type=jnp.float32
    )


def _q(ea, w, blk, e_out):
    e, ed = ea.shape
    d = w.shape[1]
    last = e // blk - 1
    return pl.pallas_call(
        _q_body,
        grid=(e_out // blk,),
        in_specs=[
            pl.BlockSpec((blk, ed), lambda i: (jnp.minimum(i, last), 0)),
            pl.BlockSpec((ed, d), lambda i: (0, 0)),
        ],
        out_specs=pl.BlockSpec((blk, d), lambda i: (i, 0)),
        out_shape=jax.ShapeDtypeStruct((e_out, d), jnp.float32),
    )(ea, w)


def _post_body(p_ref, w_ref, b_ref, t_ref, st_ref, *, nsteps):
    i = pl.program_id(0)
    p = p_ref[0] + p_ref[1]
    t = jnp.dot(p, w_ref[...], preferred_element_type=jnp.float32) + b_ref[...]
    t_ref[...] = t

    @pl.when(i == 0)
    def _():
        st_ref[...] = jnp.zeros_like(st_ref)

    st_ref[0:1, :] += jnp.sum(t, axis=0, keepdims=True)
    st_ref[1:2, :] += jnp.sum(t * t, axis=0, keepdims=True)


def _post(parts, w, b, blk, n):
    d = parts.shape[2]
    return pl.pallas_call(
        functools.partial(_post_body, nsteps=n // blk),
        grid=(n // blk,),
        in_specs=[
            pl.BlockSpec((2, blk, d), lambda i: (0, i, 0)),
            pl.BlockSpec((d, d), lambda i: (0, 0)),
            pl.BlockSpec((1, d), lambda i: (0, 0)),
        ],
        out_specs=[
            pl.BlockSpec((blk, d), lambda i: (i, 0)),
            pl.BlockSpec((2, d), lambda i: (0, 0)),
        ],
        out_shape=[
            jax.ShapeDtypeStruct((n, d), jnp.float32),
            jax.ShapeDtypeStruct((2, d), jnp.float32),
        ],
    )(parts, w, b.reshape(1, d))


def _bn(t, st, g_row, be_row, n):
    mean = st[0:1, :] * (1.0 / n)
    var = st[1:2, :] * (1.0 / n) - mean * mean
    rstd = lax.rsqrt(var + _EPS)
    h = (t - mean) * (rstd * g_row) + be_row
    return jnp.maximum(h, 0.0)


def _mid_body(t_ref, st_ref, g_ref, be_ref, w_ref, b_ref, o_ref, *, n):
    h = _bn(t_ref[...], st_ref[...], g_ref[...], be_ref[...], n)
    o_ref[...] = (
        jnp.dot(h, w_ref[...], preferred_element_type=jnp.float32) + b_ref[...]
    )


def _mid(t, st, gamma, beta, w, b, blk):
    n, d = t.shape
    return pl.pallas_call(
        functools.partial(_mid_body, n=n),
        grid=(n // blk,),
        in_specs=[
            pl.BlockSpec((blk, d), lambda i: (i, 0)),
            pl.BlockSpec((2, d), lambda i: (0, 0)),
            pl.BlockSpec((1, d), lambda i: (0, 0)),
            pl.BlockSpec((1, d), lambda i: (0, 0)),
            pl.BlockSpec((d, d), lambda i: (0, 0)),
            pl.BlockSpec((1, d), lambda i: (0, 0)),
        ],
        out_specs=pl.BlockSpec((blk, d), lambda i: (i, 0)),
        out_shape=jax.ShapeDtypeStruct((n, d), jnp.float32),
    )(t, st, gamma.reshape(1, d), beta.reshape(1, d), w, b.reshape(1, d))


def _readout_body(t_ref, st_ref, g_ref, be_ref, b_ref, o_ref, cnt_ref, *, n, nsteps):
    i = pl.program_id(0)
    h = _bn(t_ref[...], st_ref[...], g_ref[...], be_ref[...], n)
    ids = b_ref[0, 0, :]
    blk = h.shape[0]
    onehot = (
        ids[:, None] == lax.broadcasted_iota(jnp.int32, (blk, _G), 1)
    ).astype(jnp.float32)
    sums = lax.dot_general(
        onehot, h, (((0,), (0,)), ((), ())), preferred_element_type=jnp.float32
    )
    cnts = jnp.sum(onehot, axis=0, keepdims=True)

    @pl.when(i == 0)
    def _():
        o_ref[...] = jnp.zeros_like(o_ref)
        cnt_ref[...] = jnp.zeros_like(cnt_ref)

    o_ref[...] += sums
    cnt_ref[0:1, :] += cnts

    @pl.when(i == nsteps - 1)
    def _():
        o_ref[...] = o_ref[...] / jnp.maximum(cnt_ref[0:1, :], 1.0).T


def _readout(t, st, gamma, beta, batch, blk):
    n, d = t.shape
    nsteps = n // blk
    batch_r = batch.reshape(nsteps, 1, blk)
    return pl.pallas_call(
        functools.partial(_readout_body, n=n, nsteps=nsteps),
        grid=(nsteps,),
        in_specs=[
            pl.BlockSpec((blk, d), lambda i: (i, 0)),
            pl.BlockSpec((2, d), lambda i: (0, 0)),
            pl.BlockSpec((1, d), lambda i: (0, 0)),
            pl.BlockSpec((1, d), lambda i: (0, 0)),
            pl.BlockSpec((1, 1, blk), lambda i: (i, 0, 0)),
        ],
        out_specs=pl.BlockSpec((_G, d), lambda i: (0, 0)),
        out_shape=jax.ShapeDtypeStruct((_G, d), jnp.float32),
        scratch_shapes=[pltpu.VMEM((8, _G), jnp.float32)],
    )(t, st, gamma.reshape(1, d), beta.reshape(1, d), batch_r)


_SLOTS = 3
_ISLOTS = 4


_GROUPS0 = 13


def _edge_stage(p, q, src_pad, dst_pad, n_nodes):
    n, d = p.shape
    nw = _NC * _NS
    per_w = src_pad.shape[0] // nw
    n_ch = per_w // _CH
    unroll = _SLOTS * _ISLOTS
    groups_tot = 2 * n_ch // unroll
    g0, g1 = _GROUPS0, groups_tot - _GROUPS0
    n0, n1 = g0 * unroll, g1 * unroll
    pw0, pw1 = n0 * _CH, n1 * _CH
    n_rows_pad = -(-(n_nodes + 1) // (8 * _NS)) * (8 * _NS)
    zshare = n_rows_pad // _NS
    mesh = plsc.VectorSubcoreMesh(core_axis_name="c", subcore_axis_name="s")

    @functools.partial(
        pl.kernel,
        out_type=jax.ShapeDtypeStruct((_NC, n_rows_pad, d), jnp.float32),
        mesh=mesh,
        scratch_types=[
            pltpu.VMEM((_ISLOTS, _CH), jnp.int32),
            pltpu.VMEM((_ISLOTS, _CH), jnp.int32),
            pltpu.VMEM((_SLOTS, _CH, d), jnp.float32),
            pltpu.VMEM((_SLOTS, _CH, d), jnp.float32),
            pltpu.VMEM_SHARED((n_rows_pad, d), jnp.float32),
            pltpu.SemaphoreType.DMA,
            [pltpu.SemaphoreType.DMA] * _ISLOTS,
            [pltpu.SemaphoreType.DMA] * _SLOTS,
            [pltpu.SemaphoreType.DMA] * _SLOTS,
            [pltpu.SemaphoreType.DMA] * _SLOTS,
        ],
    )
    def k(p_hbm, q_hbm, src_hbm, dst_hbm, out_hbm,
          srcs, dsts, rows, qv, aggr_sh, zsem, isems, gsems, qsems, ssems):
        c = lax.axis_index("c")
        s = lax.axis_index("s")
        base_e = c * _NS * pw0 + s * pw0 + c * s * (pw1 - pw0)
        n_ch_c = n0 + c * (n1 - n0)
        groups_c = g0 + c * (g1 - g0)

        def zrow(r, _):
            for j in range(d // _LANES):
                rows[0, r, pl.ds(j * _LANES, _LANES)] = jnp.zeros(
                    (_LANES,), jnp.float32
                )
            return 0
        lax.fori_loop(0, _CH, zrow, 0)
        zbase = s * zshare
        zoff = 0
        zcopies = []
        while zoff < zshare:
            step = min(_CH, zshare - zoff)
            zcopies.append(pltpu.async_copy(
                rows.at[0, pl.ds(0, step)],
                aggr_sh.at[pl.ds(zbase + zoff, step)],
                zsem,
            ))
            zoff += step
        for cpy in zcopies:
            cpy.wait()
        plsc.subcore_barrier()

        def fetch_idx(j, ib):
            off = base_e + j * _CH
            pltpu.async_copy(src_hbm.at[pl.ds(off, _CH)], srcs.at[ib], isems[ib])
            pltpu.async_copy(dst_hbm.at[pl.ds(off, _CH)], dsts.at[ib], isems[ib])

        def wait_idx(ib):
            pltpu.make_async_copy(src_hbm.at[pl.ds(0, _CH)], srcs.at[ib],
                                  isems[ib]).wait()
            pltpu.make_async_copy(dst_hbm.at[pl.ds(0, _CH)], dsts.at[ib],
                                  isems[ib]).wait()

        def fetch(j, b, ib):
            pltpu.async_copy(p_hbm.at[srcs.at[ib]], rows.at[b], gsems[b])
            pltpu.async_copy(
                q_hbm.at[pl.ds(base_e + j * _CH, _CH)], qv.at[b], qsems[b]
            )

        def drain(b):
            pltpu.make_async_copy(
                rows.at[b], aggr_sh.at[dsts.at[0]], ssems[b]
            ).wait()

        for j in range(_ISLOTS - 1):
            fetch_idx(j, j % _ISLOTS)
        for j in range(2):
            wait_idx(j % _ISLOTS)
            fetch(j, j % _SLOTS, j % _ISLOTS)

        def group(g, _):
            for b in range(unroll):
                j = g * unroll + b
                ib = b % _ISLOTS
                b = b % _SLOTS
                pltpu.make_async_copy(
                    p_hbm.at[srcs.at[0]], rows.at[b], gsems[b]
                ).wait()
                pltpu.make_async_copy(
                    q_hbm.at[pl.ds(0, _CH)], qv.at[b], qsems[b]
                ).wait()

                def row(r, _):
                    for jj in range(d // _LANES):
                        sl = pl.ds(jj * _LANES, _LANES)
                        rows[b, r, sl] = jnp.maximum(
                            rows[b, r, sl] + qv[b, r, sl], 0.0
                        )
                    return 0
                lax.fori_loop(0, _CH, row, 0)

                pltpu.async_copy(
                    rows.at[b], aggr_sh.at[dsts.at[ib]], ssems[b],
                    add=True,
                )

                bn2 = (b + 2) % _SLOTS
                @pl.when(j >= 1)
                def _():
                    drain(bn2)

                @pl.when(j + 3 < n_ch_c)
                def _():
                    fetch_idx(j + 3, (ib + 3) % _ISLOTS)

                @pl.when(j + 2 < n_ch_c)
                def _():
                    wait_idx((ib + 2) % _ISLOTS)
                    fetch(j + 2, bn2, (ib + 2) % _ISLOTS)
            return 0

        lax.fori_loop(0, groups_c, group, 0)
        drain((n0 - 1) % _SLOTS)
        plsc.subcore_barrier()

        obase = s * zshare
        pltpu.sync_copy(
            aggr_sh.at[pl.ds(obase, zshare)],
            out_hbm.at[c, pl.ds(obase, zshare)],
        )

    return k(p, q, src_pad, dst_pad)


def kernel(x, edge_index, edge_attr, batch,
           W_pre0, b_pre0, W_post0, b_post0, gamma0, beta0,
           W_pre1, b_pre1, W_post1, b_post1, gamma1, beta1):
    n, d = x.shape
    e = edge_index.shape[1]
    nw = _NC * _NS
    chunk = nw * _CH * _SLOTS * _ISLOTS
    e_pad = -(-e // chunk) * chunk
    per_w = e_pad // nw
    n_ch = per_w // _CH

    src_pad = jnp.pad(edge_index[0], (0, e_pad - e))
    dst_pad = jnp.pad(edge_index[1], (0, e_pad - e), constant_values=n)

    blk_n = 1000 if n % 1000 == 0 else n
    blk_e = e
    for cand in (4000, 3200, 2560, 2500, 2000, 1600):
        if e % cand == 0:
            blk_e = cand
            break
    e_out = -(-e_pad // blk_e) * blk_e

    q0 = _q(edge_attr, W_pre0[d:], blk_e, e_out)
    p0 = _pre(x, W_pre0[:d], b_pre0, blk_n)
    parts0 = _edge_stage(p0, q0, src_pad, dst_pad, n)

    q1 = _q(edge_attr, W_pre1[d:], blk_e, e_out)

    t0, st0 = _post(parts0, W_post0, b_post0, blk_n, n)
    p1 = _mid(t0, st0, gamma0, beta0, W_pre1[:d], b_pre1, blk_n)
    parts1 = _edge_stage(p1, q1, src_pad, dst_pad, n)
    t1, st1 = _post(parts1, W_post1, b_post1, blk_n, n)

    return _readout(t1, st1, gamma1, beta1, batch, blk_n)


# speedup vs baseline: 1.0413x; 1.0413x over previous
import functools

import jax
import jax.numpy as jnp
import numpy as np
from jax import lax
from jax.experimental import pallas as pl
from jax.experimental.pallas import tpu as pltpu
from jax.experimental.pallas import tpu_sc as plsc

_G = 128
_EPS = 1e-5

_NC = 2
_NS = 16
_CH = 56
_LANES = 16


def _pre_body(x_ref, w_ref, b_ref, o_ref):
    o_ref[...] = (
        jnp.dot(x_ref[...], w_ref[...], preferred_element_type=jnp.float32)
        + b_ref[...]
    )


def _pre(x, w, b, blk):
    n, d = x.shape
    return pl.pallas_call(
        _pre_body,
        grid=(n // blk,),
        in_specs=[
            pl.BlockSpec((blk, d), lambda i: (i, 0)),
            pl.BlockSpec((d, d), lambda i: (0, 0)),
            pl.BlockSpec((1, d), lambda i: (0, 0)),
        ],
        out_specs=pl.BlockSpec((blk, d), lambda i: (i, 0)),
        out_shape=jax.ShapeDtypeStruct((n, d), jnp.float32),
    )(x, w, b.reshape(1, d))


def _q_body(ea_ref, w_ref, q_ref):
    q_ref[...] = jnp.dot(
        ea_ref[...], w_ref[...], preferred_element_type=jnp.float32
    )


def _q(ea, w, blk, e_out):
    e, ed = ea.shape
    d = w.shape[1]
    last = e // blk - 1
    return pl.pallas_call(
        _q_body,
        grid=(e_out // blk,),
        in_specs=[
            pl.BlockSpec((blk, ed), lambda i: (jnp.minimum(i, last), 0)),
            pl.BlockSpec((ed, d), lambda i: (0, 0)),
        ],
        out_specs=pl.BlockSpec((blk, d), lambda i: (i, 0)),
        out_shape=jax.ShapeDtypeStruct((e_out, d), jnp.float32),
    )(ea, w)


def _post_body(p_ref, w_ref, b_ref, t_ref, st_ref, *, nsteps):
    i = pl.program_id(0)
    p = p_ref[0] + p_ref[1]
    t = jnp.dot(p, w_ref[...], preferred_element_type=jnp.float32) + b_ref[...]
    t_ref[...] = t

    @pl.when(i == 0)
    def _():
        st_ref[...] = jnp.zeros_like(st_ref)

    st_ref[0:1, :] += jnp.sum(t, axis=0, keepdims=True)
    st_ref[1:2, :] += jnp.sum(t * t, axis=0, keepdims=True)


def _post(parts, w, b, blk, n):
    d = parts.shape[2]
    return pl.pallas_call(
        functools.partial(_post_body, nsteps=n // blk),
        grid=(n // blk,),
        in_specs=[
            pl.BlockSpec((2, blk, d), lambda i: (0, i, 0)),
            pl.BlockSpec((d, d), lambda i: (0, 0)),
            pl.BlockSpec((1, d), lambda i: (0, 0)),
        ],
        out_specs=[
            pl.BlockSpec((blk, d), lambda i: (i, 0)),
            pl.BlockSpec((2, d), lambda i: (0, 0)),
        ],
        out_shape=[
            jax.ShapeDtypeStruct((n, d), jnp.float32),
            jax.ShapeDtypeStruct((2, d), jnp.float32),
        ],
    )(parts, w, b.reshape(1, d))


def _bn(t, st, g_row, be_row, n):
    mean = st[0:1, :] * (1.0 / n)
    var = st[1:2, :] * (1.0 / n) - mean * mean
    rstd = lax.rsqrt(var + _EPS)
    h = (t - mean) * (rstd * g_row) + be_row
    return jnp.maximum(h, 0.0)


def _mid_body(t_ref, st_ref, g_ref, be_ref, w_ref, b_ref, o_ref, *, n):
    h = _bn(t_ref[...], st_ref[...], g_ref[...], be_ref[...], n)
    o_ref[...] = (
        jnp.dot(h, w_ref[...], preferred_element_type=jnp.float32) + b_ref[...]
    )


def _mid(t, st, gamma, beta, w, b, blk):
    n, d = t.shape
    return pl.pallas_call(
        functools.partial(_mid_body, n=n),
        grid=(n // blk,),
        in_specs=[
            pl.BlockSpec((blk, d), lambda i: (i, 0)),
            pl.BlockSpec((2, d), lambda i: (0, 0)),
            pl.BlockSpec((1, d), lambda i: (0, 0)),
            pl.BlockSpec((1, d), lambda i: (0, 0)),
            pl.BlockSpec((d, d), lambda i: (0, 0)),
            pl.BlockSpec((1, d), lambda i: (0, 0)),
        ],
        out_specs=pl.BlockSpec((blk, d), lambda i: (i, 0)),
        out_shape=jax.ShapeDtypeStruct((n, d), jnp.float32),
    )(t, st, gamma.reshape(1, d), beta.reshape(1, d), w, b.reshape(1, d))


def _readout_body(t_ref, st_ref, g_ref, be_ref, b_ref, o_ref, cnt_ref, *, n, nsteps):
    i = pl.program_id(0)
    h = _bn(t_ref[...], st_ref[...], g_ref[...], be_ref[...], n)
    ids = b_ref[0, 0, :]
    blk = h.shape[0]
    onehot = (
        ids[:, None] == lax.broadcasted_iota(jnp.int32, (blk, _G), 1)
    ).astype(jnp.float32)
    sums = lax.dot_general(
        onehot, h, (((0,), (0,)), ((), ())), preferred_element_type=jnp.float32
    )
    cnts = jnp.sum(onehot, axis=0, keepdims=True)

    @pl.when(i == 0)
    def _():
        o_ref[...] = jnp.zeros_like(o_ref)
        cnt_ref[...] = jnp.zeros_like(cnt_ref)

    o_ref[...] += sums
    cnt_ref[0:1, :] += cnts

    @pl.when(i == nsteps - 1)
    def _():
        o_ref[...] = o_ref[...] / jnp.maximum(cnt_ref[0:1, :], 1.0).T


def _readout(t, st, gamma, beta, batch, blk):
    n, d = t.shape
    nsteps = n // blk
    batch_r = batch.reshape(nsteps, 1, blk)
    return pl.pallas_call(
        functools.partial(_readout_body, n=n, nsteps=nsteps),
        grid=(nsteps,),
        in_specs=[
            pl.BlockSpec((blk, d), lambda i: (i, 0)),
            pl.BlockSpec((2, d), lambda i: (0, 0)),
            pl.BlockSpec((1, d), lambda i: (0, 0)),
            pl.BlockSpec((1, d), lambda i: (0, 0)),
            pl.BlockSpec((1, 1, blk), lambda i: (i, 0, 0)),
        ],
        out_specs=pl.BlockSpec((_G, d), lambda i: (0, 0)),
        out_shape=jax.ShapeDtypeStruct((_G, d), jnp.float32),
        scratch_shapes=[pltpu.VMEM((8, _G), jnp.float32)],
    )(t, st, gamma.reshape(1, d), beta.reshape(1, d), batch_r)


_SLOTS = 3
_ISLOTS = 4


_GROUPS0 = 17


def _edge_stage(p, q, src_pad, dst_pad, n_nodes):
    n, d = p.shape
    nw = _NC * _NS
    per_w = src_pad.shape[0] // nw
    n_ch = per_w // _CH
    unroll = _SLOTS * _ISLOTS
    groups_tot = 2 * n_ch // unroll
    g0, g1 = _GROUPS0, groups_tot - _GROUPS0
    n0, n1 = g0 * unroll, g1 * unroll
    pw0, pw1 = n0 * _CH, n1 * _CH
    n_rows_pad = -(-(n_nodes + 1) // (8 * _NS)) * (8 * _NS)
    zshare = n_rows_pad // _NS
    mesh = plsc.VectorSubcoreMesh(core_axis_name="c", subcore_axis_name="s")

    @functools.partial(
        pl.kernel,
        out_type=jax.ShapeDtypeStruct((_NC, n_rows_pad, d), jnp.float32),
        mesh=mesh,
        scratch_types=[
            pltpu.VMEM((_ISLOTS, _CH), jnp.int32),
            pltpu.VMEM((_ISLOTS, _CH), jnp.int32),
            pltpu.VMEM((_SLOTS, _CH, d), jnp.float32),
            pltpu.VMEM((_SLOTS, _CH, d), jnp.float32),
            pltpu.VMEM_SHARED((n_rows_pad, d), jnp.float32),
            pltpu.SemaphoreType.DMA,
            [pltpu.SemaphoreType.DMA] * _ISLOTS,
            [pltpu.SemaphoreType.DMA] * _SLOTS,
            [pltpu.SemaphoreType.DMA] * _SLOTS,
            [pltpu.SemaphoreType.DMA] * _SLOTS,
        ],
    )
    def k(p_hbm, q_hbm, src_hbm, dst_hbm, out_hbm,
          srcs, dsts, rows, qv, aggr_sh, zsem, isems, gsems, qsems, ssems):
        c = lax.axis_index("c")
        s = lax.axis_index("s")
        base_e = c * _NS * pw0 + s * pw0 + c * s * (pw1 - pw0)
        n_ch_c = n0 + c * (n1 - n0)
        groups_c = g0 + c * (g1 - g0)

        def zrow(r, _):
            for j in range(d // _LANES):
                rows[0, r, pl.ds(j * _LANES, _LANES)] = jnp.zeros(
                    (_LANES,), jnp.float32
                )
            return 0
        lax.fori_loop(0, _CH, zrow, 0)
        zbase = s * zshare
        zoff = 0
        zcopies = []
        while zoff < zshare:
            step = min(_CH, zshare - zoff)
            zcopies.append(pltpu.async_copy(
                rows.at[0, pl.ds(0, step)],
                aggr_sh.at[pl.ds(zbase + zoff, step)],
                zsem,
            ))
            zoff += step
        for cpy in zcopies:
            cpy.wait()
        plsc.subcore_barrier()

        def fetch_idx(j, ib):
            off = base_e + j * _CH
            pltpu.async_copy(src_hbm.at[pl.ds(off, _CH)], srcs.at[ib], isems[ib])
            pltpu.async_copy(dst_hbm.at[pl.ds(off, _CH)], dsts.at[ib], isems[ib])

        def wait_idx(ib):
            pltpu.make_async_copy(src_hbm.at[pl.ds(0, _CH)], srcs.at[ib],
                                  isems[ib]).wait()
            pltpu.make_async_copy(dst_hbm.at[pl.ds(0, _CH)], dsts.at[ib],
                                  isems[ib]).wait()

        def fetch(j, b, ib):
            pltpu.async_copy(p_hbm.at[srcs.at[ib]], rows.at[b], gsems[b])
            pltpu.async_copy(
                q_hbm.at[pl.ds(base_e + j * _CH, _CH)], qv.at[b], qsems[b]
            )

        def drain(b):
            pltpu.make_async_copy(
                rows.at[b], aggr_sh.at[dsts.at[0]], ssems[b]
            ).wait()

        for j in range(_ISLOTS - 1):
            fetch_idx(j, j % _ISLOTS)
        for j in range(2):
            wait_idx(j % _ISLOTS)
            fetch(j, j % _SLOTS, j % _ISLOTS)

        def group(g, _):
            for b in range(unroll):
                j = g * unroll + b
                ib = b % _ISLOTS
                b = b % _SLOTS
                pltpu.make_async_copy(
                    p_hbm.at[srcs.at[0]], rows.at[b], gsems[b]
                ).wait()
                pltpu.make_async_copy(
                    q_hbm.at[pl.ds(0, _CH)], qv.at[b], qsems[b]
                ).wait()

                def row(r, _):
                    for jj in range(d // _LANES):
                        sl = pl.ds(jj * _LANES, _LANES)
                        rows[b, r, sl] = jnp.maximum(
                            rows[b, r, sl] + qv[b, r, sl], 0.0
                        )
                    return 0
                lax.fori_loop(0, _CH, row, 0)

                pltpu.async_copy(
                    rows.at[b], aggr_sh.at[dsts.at[ib]], ssems[b],
                    add=True,
                )

                bn2 = (b + 2) % _SLOTS
                @pl.when(j >= 1)
                def _():
                    drain(bn2)

                @pl.when(j + 3 < n_ch_c)
                def _():
                    fetch_idx(j + 3, (ib + 3) % _ISLOTS)

                @pl.when(j + 2 < n_ch_c)
                def _():
                    wait_idx((ib + 2) % _ISLOTS)
                    fetch(j + 2, bn2, (ib + 2) % _ISLOTS)
            return 0

        lax.fori_loop(0, groups_c, group, 0)
        drain((n0 - 1) % _SLOTS)
        plsc.subcore_barrier()

        obase = s * zshare
        pltpu.sync_copy(
            aggr_sh.at[pl.ds(obase, zshare)],
            out_hbm.at[c, pl.ds(obase, zshare)],
        )

    return k(p, q, src_pad, dst_pad)


def kernel(x, edge_index, edge_attr, batch,
           W_pre0, b_pre0, W_post0, b_post0, gamma0, beta0,
           W_pre1, b_pre1, W_post1, b_post1, gamma1, beta1):
    n, d = x.shape
    e = edge_index.shape[1]
    nw = _NC * _NS
    chunk = nw * _CH * _SLOTS * _ISLOTS
    e_pad = -(-e // chunk) * chunk
    per_w = e_pad // nw
    n_ch = per_w // _CH

    src_pad = jnp.pad(edge_index[0], (0, e_pad - e))
    dst_pad = jnp.pad(edge_index[1], (0, e_pad - e), constant_values=n)

    blk_n = 1000 if n % 1000 == 0 else n
    blk_e = e
    for cand in (4000, 3200, 2560, 2500, 2000, 1600):
        if e % cand == 0:
            blk_e = cand
            break
    e_out = -(-e_pad // blk_e) * blk_e

    q0 = _q(edge_attr, W_pre0[d:], blk_e, e_out)
    p0 = _pre(x, W_pre0[:d], b_pre0, blk_n)
    parts0 = _edge_stage(p0, q0, src_pad, dst_pad, n)

    q1 = _q(edge_attr, W_pre1[d:], blk_e, e_out)

    t0, st0 = _post(parts0, W_post0, b_post0, blk_n, n)
    p1 = _mid(t0, st0, gamma0, beta0, W_pre1[:d], b_pre1, blk_n)
    parts1 = _edge_stage(p1, q1, src_pad, dst_pad, n)
    t1, st1 = _post(parts1, W_post1, b_post1, blk_n, n)

    return _readout(t1, st1, gamma1, beta1, batch, blk_n)


# speedup vs baseline: 1.0531x; 1.0113x over previous
import functools

import jax
import jax.numpy as jnp
import numpy as np
from jax import lax
from jax.experimental import pallas as pl
from jax.experimental.pallas import tpu as pltpu
from jax.experimental.pallas import tpu_sc as plsc

_G = 128
_EPS = 1e-5

_NC = 2
_NS = 16
_CH = 56
_LANES = 16


def _pre_body(x_ref, w_ref, b_ref, o_ref):
    o_ref[...] = (
        jnp.dot(x_ref[...], w_ref[...], preferred_element_type=jnp.float32)
        + b_ref[...]
    )


def _pre(x, w, b, blk):
    n, d = x.shape
    return pl.pallas_call(
        _pre_body,
        grid=(n // blk,),
        in_specs=[
            pl.BlockSpec((blk, d), lambda i: (i, 0)),
            pl.BlockSpec((d, d), lambda i: (0, 0)),
            pl.BlockSpec((1, d), lambda i: (0, 0)),
        ],
        out_specs=pl.BlockSpec((blk, d), lambda i: (i, 0)),
        out_shape=jax.ShapeDtypeStruct((n, d), jnp.float32),
    )(x, w, b.reshape(1, d))


def _q_body(ea_ref, w_ref, q_ref):
    q_ref[...] = jnp.dot(
        ea_ref[...], w_ref[...], preferred_element_type=jnp.float32
    )


def _q(ea, w, blk, e_out):
    e, ed = ea.shape
    d = w.shape[1]
    last = e // blk - 1
    return pl.pallas_call(
        _q_body,
        grid=(e_out // blk,),
        in_specs=[
            pl.BlockSpec((blk, ed), lambda i: (jnp.minimum(i, last), 0)),
            pl.BlockSpec((ed, d), lambda i: (0, 0)),
        ],
        out_specs=pl.BlockSpec((blk, d), lambda i: (i, 0)),
        out_shape=jax.ShapeDtypeStruct((e_out, d), jnp.float32),
    )(ea, w)


def _post_body(p_ref, w_ref, b_ref, t_ref, st_ref, *, nsteps):
    i = pl.program_id(0)
    p = p_ref[0] + p_ref[1]
    t = jnp.dot(p, w_ref[...], preferred_element_type=jnp.float32) + b_ref[...]
    t_ref[...] = t

    @pl.when(i == 0)
    def _():
        st_ref[...] = jnp.zeros_like(st_ref)

    st_ref[0:1, :] += jnp.sum(t, axis=0, keepdims=True)
    st_ref[1:2, :] += jnp.sum(t * t, axis=0, keepdims=True)


def _post(parts, w, b, blk, n):
    d = parts.shape[2]
    return pl.pallas_call(
        functools.partial(_post_body, nsteps=n // blk),
        grid=(n // blk,),
        in_specs=[
            pl.BlockSpec((2, blk, d), lambda i: (0, i, 0)),
            pl.BlockSpec((d, d), lambda i: (0, 0)),
            pl.BlockSpec((1, d), lambda i: (0, 0)),
        ],
        out_specs=[
            pl.BlockSpec((blk, d), lambda i: (i, 0)),
            pl.BlockSpec((2, d), lambda i: (0, 0)),
        ],
        out_shape=[
            jax.ShapeDtypeStruct((n, d), jnp.float32),
            jax.ShapeDtypeStruct((2, d), jnp.float32),
        ],
    )(parts, w, b.reshape(1, d))


def _bn(t, st, g_row, be_row, n):
    mean = st[0:1, :] * (1.0 / n)
    var = st[1:2, :] * (1.0 / n) - mean * mean
    rstd = lax.rsqrt(var + _EPS)
    h = (t - mean) * (rstd * g_row) + be_row
    return jnp.maximum(h, 0.0)


def _mid_body(t_ref, st_ref, g_ref, be_ref, w_ref, b_ref, o_ref, *, n):
    h = _bn(t_ref[...], st_ref[...], g_ref[...], be_ref[...], n)
    o_ref[...] = (
        jnp.dot(h, w_ref[...], preferred_element_type=jnp.float32) + b_ref[...]
    )


def _mid(t, st, gamma, beta, w, b, blk):
    n, d = t.shape
    return pl.pallas_call(
        functools.partial(_mid_body, n=n),
        grid=(n // blk,),
        in_specs=[
            pl.BlockSpec((blk, d), lambda i: (i, 0)),
            pl.BlockSpec((2, d), lambda i: (0, 0)),
            pl.BlockSpec((1, d), lambda i: (0, 0)),
            pl.BlockSpec((1, d), lambda i: (0, 0)),
            pl.BlockSpec((d, d), lambda i: (0, 0)),
            pl.BlockSpec((1, d), lambda i: (0, 0)),
        ],
        out_specs=pl.BlockSpec((blk, d), lambda i: (i, 0)),
        out_shape=jax.ShapeDtypeStruct((n, d), jnp.float32),
    )(t, st, gamma.reshape(1, d), beta.reshape(1, d), w, b.reshape(1, d))


def _readout_body(t_ref, st_ref, g_ref, be_ref, b_ref, o_ref, cnt_ref, *, n, nsteps):
    i = pl.program_id(0)
    h = _bn(t_ref[...], st_ref[...], g_ref[...], be_ref[...], n)
    ids = b_ref[0, 0, :]
    blk = h.shape[0]
    onehot = (
        ids[:, None] == lax.broadcasted_iota(jnp.int32, (blk, _G), 1)
    ).astype(jnp.float32)
    sums = lax.dot_general(
        onehot, h, (((0,), (0,)), ((), ())), preferred_element_type=jnp.float32
    )
    cnts = jnp.sum(onehot, axis=0, keepdims=True)

    @pl.when(i == 0)
    def _():
        o_ref[...] = jnp.zeros_like(o_ref)
        cnt_ref[...] = jnp.zeros_like(cnt_ref)

    o_ref[...] += sums
    cnt_ref[0:1, :] += cnts

    @pl.when(i == nsteps - 1)
    def _():
        o_ref[...] = o_ref[...] / jnp.maximum(cnt_ref[0:1, :], 1.0).T


def _readout(t, st, gamma, beta, batch, blk):
    n, d = t.shape
    nsteps = n // blk
    batch_r = batch.reshape(nsteps, 1, blk)
    return pl.pallas_call(
        functools.partial(_readout_body, n=n, nsteps=nsteps),
        grid=(nsteps,),
        in_specs=[
            pl.BlockSpec((blk, d), lambda i: (i, 0)),
            pl.BlockSpec((2, d), lambda i: (0, 0)),
            pl.BlockSpec((1, d), lambda i: (0, 0)),
            pl.BlockSpec((1, d), lambda i: (0, 0)),
            pl.BlockSpec((1, 1, blk), lambda i: (i, 0, 0)),
        ],
        out_specs=pl.BlockSpec((_G, d), lambda i: (0, 0)),
        out_shape=jax.ShapeDtypeStruct((_G, d), jnp.float32),
        scratch_shapes=[pltpu.VMEM((8, _G), jnp.float32)],
    )(t, st, gamma.reshape(1, d), beta.reshape(1, d), batch_r)


_SLOTS = 3
_ISLOTS = 4


_GROUPS0 = 18


def _edge_stage(p, q, src_pad, dst_pad, n_nodes):
    n, d = p.shape
    nw = _NC * _NS
    per_w = src_pad.shape[0] // nw
    n_ch = per_w // _CH
    unroll = _SLOTS * _ISLOTS
    groups_tot = 2 * n_ch // unroll
    g0, g1 = _GROUPS0, groups_tot - _GROUPS0
    n0, n1 = g0 * unroll, g1 * unroll
    pw0, pw1 = n0 * _CH, n1 * _CH
    n_rows_pad = -(-(n_nodes + 1) // (8 * _NS)) * (8 * _NS)
    zshare = n_rows_pad // _NS
    mesh = plsc.VectorSubcoreMesh(core_axis_name="c", subcore_axis_name="s")

    @functools.partial(
        pl.kernel,
        out_type=jax.ShapeDtypeStruct((_NC, n_rows_pad, d), jnp.float32),
        mesh=mesh,
        scratch_types=[
            pltpu.VMEM((_ISLOTS, _CH), jnp.int32),
            pltpu.VMEM((_ISLOTS, _CH), jnp.int32),
            pltpu.VMEM((_SLOTS, _CH, d), jnp.float32),
            pltpu.VMEM((_SLOTS, _CH, d), jnp.float32),
            pltpu.VMEM_SHARED((n_rows_pad, d), jnp.float32),
            pltpu.SemaphoreType.DMA,
            [pltpu.SemaphoreType.DMA] * _ISLOTS,
            [pltpu.SemaphoreType.DMA] * _SLOTS,
            [pltpu.SemaphoreType.DMA] * _SLOTS,
            [pltpu.SemaphoreType.DMA] * _SLOTS,
        ],
    )
    def k(p_hbm, q_hbm, src_hbm, dst_hbm, out_hbm,
          srcs, dsts, rows, qv, aggr_sh, zsem, isems, gsems, qsems, ssems):
        c = lax.axis_index("c")
        s = lax.axis_index("s")
        base_e = c * _NS * pw0 + s * pw0 + c * s * (pw1 - pw0)
        n_ch_c = n0 + c * (n1 - n0)
        groups_c = g0 + c * (g1 - g0)

        def zrow(r, _):
            for j in range(d // _LANES):
                rows[0, r, pl.ds(j * _LANES, _LANES)] = jnp.zeros(
                    (_LANES,), jnp.float32
                )
            return 0
        lax.fori_loop(0, _CH, zrow, 0)
        zbase = s * zshare
        zoff = 0
        zcopies = []
        while zoff < zshare:
            step = min(_CH, zshare - zoff)
            zcopies.append(pltpu.async_copy(
                rows.at[0, pl.ds(0, step)],
                aggr_sh.at[pl.ds(zbase + zoff, step)],
                zsem,
            ))
            zoff += step
        for cpy in zcopies:
            cpy.wait()
        plsc.subcore_barrier()

        def fetch_idx(j, ib):
            off = base_e + j * _CH
            pltpu.async_copy(src_hbm.at[pl.ds(off, _CH)], srcs.at[ib], isems[ib])
            pltpu.async_copy(dst_hbm.at[pl.ds(off, _CH)], dsts.at[ib], isems[ib])

        def wait_idx(ib):
            pltpu.make_async_copy(src_hbm.at[pl.ds(0, _CH)], srcs.at[ib],
                                  isems[ib]).wait()
            pltpu.make_async_copy(dst_hbm.at[pl.ds(0, _CH)], dsts.at[ib],
                                  isems[ib]).wait()

        def fetch(j, b, ib):
            pltpu.async_copy(p_hbm.at[srcs.at[ib]], rows.at[b], gsems[b])
            pltpu.async_copy(
                q_hbm.at[pl.ds(base_e + j * _CH, _CH)], qv.at[b], qsems[b]
            )

        def drain(b):
            pltpu.make_async_copy(
                rows.at[b], aggr_sh.at[dsts.at[0]], ssems[b]
            ).wait()

        for j in range(_ISLOTS - 1):
            fetch_idx(j, j % _ISLOTS)
        for j in range(2):
            wait_idx(j % _ISLOTS)
            fetch(j, j % _SLOTS, j % _ISLOTS)

        def group(g, _):
            for b in range(unroll):
                j = g * unroll + b
                ib = b % _ISLOTS
                b = b % _SLOTS
                pltpu.make_async_copy(
                    p_hbm.at[srcs.at[0]], rows.at[b], gsems[b]
                ).wait()
                pltpu.make_async_copy(
                    q_hbm.at[pl.ds(0, _CH)], qv.at[b], qsems[b]
                ).wait()

                def row(r, _):
                    for jj in range(d // _LANES):
                        sl = pl.ds(jj * _LANES, _LANES)
                        rows[b, r, sl] = jnp.maximum(
                            rows[b, r, sl] + qv[b, r, sl], 0.0
                        )
                    return 0
                lax.fori_loop(0, _CH, row, 0)

                pltpu.async_copy(
                    rows.at[b], aggr_sh.at[dsts.at[ib]], ssems[b],
                    add=True,
                )

                bn2 = (b + 2) % _SLOTS
                @pl.when(j >= 1)
                def _():
                    drain(bn2)

                @pl.when(j + 3 < n_ch_c)
                def _():
                    fetch_idx(j + 3, (ib + 3) % _ISLOTS)

                @pl.when(j + 2 < n_ch_c)
                def _():
                    wait_idx((ib + 2) % _ISLOTS)
                    fetch(j + 2, bn2, (ib + 2) % _ISLOTS)
            return 0

        lax.fori_loop(0, groups_c, group, 0)
        drain((n0 - 1) % _SLOTS)
        plsc.subcore_barrier()

        obase = s * zshare
        pltpu.sync_copy(
            aggr_sh.at[pl.ds(obase, zshare)],
            out_hbm.at[c, pl.ds(obase, zshare)],
        )

    return k(p, q, src_pad, dst_pad)


def kernel(x, edge_index, edge_attr, batch,
           W_pre0, b_pre0, W_post0, b_post0, gamma0, beta0,
           W_pre1, b_pre1, W_post1, b_post1, gamma1, beta1):
    n, d = x.shape
    e = edge_index.shape[1]
    nw = _NC * _NS
    chunk = nw * _CH * _SLOTS * _ISLOTS
    e_pad = -(-e // chunk) * chunk
    per_w = e_pad // nw
    n_ch = per_w // _CH

    src_pad = jnp.pad(edge_index[0], (0, e_pad - e))
    dst_pad = jnp.pad(edge_index[1], (0, e_pad - e), constant_values=n)

    blk_n = 1000 if n % 1000 == 0 else n
    blk_e = e
    for cand in (4000, 3200, 2560, 2500, 2000, 1600):
        if e % cand == 0:
            blk_e = cand
            break
    e_out = -(-e_pad // blk_e) * blk_e

    q0 = _q(edge_attr, W_pre0[d:], blk_e, e_out)
    p0 = _pre(x, W_pre0[:d], b_pre0, blk_n)
    parts0 = _edge_stage(p0, q0, src_pad, dst_pad, n)

    q1 = _q(edge_attr, W_pre1[d:], blk_e, e_out)

    t0, st0 = _post(parts0, W_post0, b_post0, blk_n, n)
    p1 = _mid(t0, st0, gamma0, beta0, W_pre1[:d], b_pre1, blk_n)
    parts1 = _edge_stage(p1, q1, src_pad, dst_pad, n)
    t1, st1 = _post(parts1, W_post1, b_post1, blk_n, n)

    return _readout(t1, st1, gamma1, beta1, batch, blk_n)


# speedup vs baseline: 1.0612x; 1.0077x over previous
import functools

import jax
import jax.numpy as jnp
import numpy as np
from jax import lax
from jax.experimental import pallas as pl
from jax.experimental.pallas import tpu as pltpu
from jax.experimental.pallas import tpu_sc as plsc

_G = 128
_EPS = 1e-5

_NC = 2
_NS = 16
_CH = 56
_LANES = 16


def _pre_body(x_ref, w_ref, b_ref, o_ref):
    o_ref[...] = (
        jnp.dot(x_ref[...], w_ref[...], preferred_element_type=jnp.float32)
        + b_ref[...]
    )


def _pre(x, w, b, blk):
    n, d = x.shape
    return pl.pallas_call(
        _pre_body,
        grid=(n // blk,),
        in_specs=[
            pl.BlockSpec((blk, d), lambda i: (i, 0)),
            pl.BlockSpec((d, d), lambda i: (0, 0)),
            pl.BlockSpec((1, d), lambda i: (0, 0)),
        ],
        out_specs=pl.BlockSpec((blk, d), lambda i: (i, 0)),
        out_shape=jax.ShapeDtypeStruct((n, d), jnp.float32),
    )(x, w, b.reshape(1, d))


def _q_body(ea_ref, w_ref, q_ref):
    q_ref[...] = jnp.dot(
        ea_ref[...], w_ref[...], preferred_element_type=jnp.float32
    )


def _q(ea, w, blk, e_out):
    e, ed = ea.shape
    d = w.shape[1]
    last = e // blk - 1
    return pl.pallas_call(
        _q_body,
        grid=(e_out // blk,),
        in_specs=[
            pl.BlockSpec((blk, ed), lambda i: (jnp.minimum(i, last), 0)),
            pl.BlockSpec((ed, d), lambda i: (0, 0)),
        ],
        out_specs=pl.BlockSpec((blk, d), lambda i: (i, 0)),
        out_shape=jax.ShapeDtypeStruct((e_out, d), jnp.float32),
    )(ea, w)


def _post_body(p_ref, w_ref, b_ref, t_ref, st_ref, *, nsteps):
    i = pl.program_id(0)
    p = p_ref[0] + p_ref[1]
    t = jnp.dot(p, w_ref[...], preferred_element_type=jnp.float32) + b_ref[...]
    t_ref[...] = t

    @pl.when(i == 0)
    def _():
        st_ref[...] = jnp.zeros_like(st_ref)

    st_ref[0:1, :] += jnp.sum(t, axis=0, keepdims=True)
    st_ref[1:2, :] += jnp.sum(t * t, axis=0, keepdims=True)


def _post(parts, w, b, blk, n):
    d = parts.shape[2]
    return pl.pallas_call(
        functools.partial(_post_body, nsteps=n // blk),
        grid=(n // blk,),
        in_specs=[
            pl.BlockSpec((2, blk, d), lambda i: (0, i, 0)),
            pl.BlockSpec((d, d), lambda i: (0, 0)),
            pl.BlockSpec((1, d), lambda i: (0, 0)),
        ],
        out_specs=[
            pl.BlockSpec((blk, d), lambda i: (i, 0)),
            pl.BlockSpec((2, d), lambda i: (0, 0)),
        ],
        out_shape=[
            jax.ShapeDtypeStruct((n, d), jnp.float32),
            jax.ShapeDtypeStruct((2, d), jnp.float32),
        ],
    )(parts, w, b.reshape(1, d))


def _bn(t, st, g_row, be_row, n):
    mean = st[0:1, :] * (1.0 / n)
    var = st[1:2, :] * (1.0 / n) - mean * mean
    rstd = lax.rsqrt(var + _EPS)
    h = (t - mean) * (rstd * g_row) + be_row
    return jnp.maximum(h, 0.0)


def _mid_body(t_ref, st_ref, g_ref, be_ref, w_ref, b_ref, o_ref, *, n):
    h = _bn(t_ref[...], st_ref[...], g_ref[...], be_ref[...], n)
    o_ref[...] = (
        jnp.dot(h, w_ref[...], preferred_element_type=jnp.float32) + b_ref[...]
    )


def _mid(t, st, gamma, beta, w, b, blk):
    n, d = t.shape
    return pl.pallas_call(
        functools.partial(_mid_body, n=n),
        grid=(n // blk,),
        in_specs=[
            pl.BlockSpec((blk, d), lambda i: (i, 0)),
            pl.BlockSpec((2, d), lambda i: (0, 0)),
            pl.BlockSpec((1, d), lambda i: (0, 0)),
            pl.BlockSpec((1, d), lambda i: (0, 0)),
            pl.BlockSpec((d, d), lambda i: (0, 0)),
            pl.BlockSpec((1, d), lambda i: (0, 0)),
        ],
        out_specs=pl.BlockSpec((blk, d), lambda i: (i, 0)),
        out_shape=jax.ShapeDtypeStruct((n, d), jnp.float32),
    )(t, st, gamma.reshape(1, d), beta.reshape(1, d), w, b.reshape(1, d))


def _readout_body(t_ref, st_ref, g_ref, be_ref, b_ref, o_ref, cnt_ref, *, n, nsteps):
    i = pl.program_id(0)
    h = _bn(t_ref[...], st_ref[...], g_ref[...], be_ref[...], n)
    ids = b_ref[0, 0, :]
    blk = h.shape[0]
    onehot = (
        ids[:, None] == lax.broadcasted_iota(jnp.int32, (blk, _G), 1)
    ).astype(jnp.float32)
    sums = lax.dot_general(
        onehot, h, (((0,), (0,)), ((), ())), preferred_element_type=jnp.float32
    )
    cnts = jnp.sum(onehot, axis=0, keepdims=True)

    @pl.when(i == 0)
    def _():
        o_ref[...] = jnp.zeros_like(o_ref)
        cnt_ref[...] = jnp.zeros_like(cnt_ref)

    o_ref[...] += sums
    cnt_ref[0:1, :] += cnts

    @pl.when(i == nsteps - 1)
    def _():
        o_ref[...] = o_ref[...] / jnp.maximum(cnt_ref[0:1, :], 1.0).T


def _readout(t, st, gamma, beta, batch, blk):
    n, d = t.shape
    nsteps = n // blk
    batch_r = batch.reshape(nsteps, 1, blk)
    return pl.pallas_call(
        functools.partial(_readout_body, n=n, nsteps=nsteps),
        grid=(nsteps,),
        in_specs=[
            pl.BlockSpec((blk, d), lambda i: (i, 0)),
            pl.BlockSpec((2, d), lambda i: (0, 0)),
            pl.BlockSpec((1, d), lambda i: (0, 0)),
            pl.BlockSpec((1, d), lambda i: (0, 0)),
            pl.BlockSpec((1, 1, blk), lambda i: (i, 0, 0)),
        ],
        out_specs=pl.BlockSpec((_G, d), lambda i: (0, 0)),
        out_shape=jax.ShapeDtypeStruct((_G, d), jnp.float32),
        scratch_shapes=[pltpu.VMEM((8, _G), jnp.float32)],
    )(t, st, gamma.reshape(1, d), beta.reshape(1, d), batch_r)


_SLOTS = 3
_ISLOTS = 4


_GROUPS0 = 19


def _edge_stage(p, q, src_pad, dst_pad, n_nodes):
    n, d = p.shape
    nw = _NC * _NS
    per_w = src_pad.shape[0] // nw
    n_ch = per_w // _CH
    unroll = _SLOTS * _ISLOTS
    groups_tot = 2 * n_ch // unroll
    g0, g1 = _GROUPS0, groups_tot - _GROUPS0
    n0, n1 = g0 * unroll, g1 * unroll
    pw0, pw1 = n0 * _CH, n1 * _CH
    n_rows_pad = -(-(n_nodes + 1) // (8 * _NS)) * (8 * _NS)
    zshare = n_rows_pad // _NS
    mesh = plsc.VectorSubcoreMesh(core_axis_name="c", subcore_axis_name="s")

    @functools.partial(
        pl.kernel,
        out_type=jax.ShapeDtypeStruct((_NC, n_rows_pad, d), jnp.float32),
        mesh=mesh,
        scratch_types=[
            pltpu.VMEM((_ISLOTS, _CH), jnp.int32),
            pltpu.VMEM((_ISLOTS, _CH), jnp.int32),
            pltpu.VMEM((_SLOTS, _CH, d), jnp.float32),
            pltpu.VMEM((_SLOTS, _CH, d), jnp.float32),
            pltpu.VMEM_SHARED((n_rows_pad, d), jnp.float32),
            pltpu.SemaphoreType.DMA,
            [pltpu.SemaphoreType.DMA] * _ISLOTS,
            [pltpu.SemaphoreType.DMA] * _SLOTS,
            [pltpu.SemaphoreType.DMA] * _SLOTS,
            [pltpu.SemaphoreType.DMA] * _SLOTS,
        ],
    )
    def k(p_hbm, q_hbm, src_hbm, dst_hbm, out_hbm,
          srcs, dsts, rows, qv, aggr_sh, zsem, isems, gsems, qsems, ssems):
        c = lax.axis_index("c")
        s = lax.axis_index("s")
        base_e = c * _NS * pw0 + s * pw0 + c * s * (pw1 - pw0)
        n_ch_c = n0 + c * (n1 - n0)
        groups_c = g0 + c * (g1 - g0)

        def zrow(r, _):
            for j in range(d // _LANES):
                rows[0, r, pl.ds(j * _LANES, _LANES)] = jnp.zeros(
                    (_LANES,), jnp.float32
                )
            return 0
        lax.fori_loop(0, _CH, zrow, 0)
        zbase = s * zshare
        zoff = 0
        zcopies = []
        while zoff < zshare:
            step = min(_CH, zshare - zoff)
            zcopies.append(pltpu.async_copy(
                rows.at[0, pl.ds(0, step)],
                aggr_sh.at[pl.ds(zbase + zoff, step)],
                zsem,
            ))
            zoff += step
        for cpy in zcopies:
            cpy.wait()
        plsc.subcore_barrier()

        def fetch_idx(j, ib):
            off = base_e + j * _CH
            pltpu.async_copy(src_hbm.at[pl.ds(off, _CH)], srcs.at[ib], isems[ib])
            pltpu.async_copy(dst_hbm.at[pl.ds(off, _CH)], dsts.at[ib], isems[ib])

        def wait_idx(ib):
            pltpu.make_async_copy(src_hbm.at[pl.ds(0, _CH)], srcs.at[ib],
                                  isems[ib]).wait()
            pltpu.make_async_copy(dst_hbm.at[pl.ds(0, _CH)], dsts.at[ib],
                                  isems[ib]).wait()

        def fetch(j, b, ib):
            pltpu.async_copy(p_hbm.at[srcs.at[ib]], rows.at[b], gsems[b])
            pltpu.async_copy(
                q_hbm.at[pl.ds(base_e + j * _CH, _CH)], qv.at[b], qsems[b]
            )

        def drain(b):
            pltpu.make_async_copy(
                rows.at[b], aggr_sh.at[dsts.at[0]], ssems[b]
            ).wait()

        for j in range(_ISLOTS - 1):
            fetch_idx(j, j % _ISLOTS)
        for j in range(2):
            wait_idx(j % _ISLOTS)
            fetch(j, j % _SLOTS, j % _ISLOTS)

        def group(g, _):
            for b in range(unroll):
                j = g * unroll + b
                ib = b % _ISLOTS
                b = b % _SLOTS
                pltpu.make_async_copy(
                    p_hbm.at[srcs.at[0]], rows.at[b], gsems[b]
                ).wait()
                pltpu.make_async_copy(
                    q_hbm.at[pl.ds(0, _CH)], qv.at[b], qsems[b]
                ).wait()

                def row(r, _):
                    for jj in range(d // _LANES):
                        sl = pl.ds(jj * _LANES, _LANES)
                        rows[b, r, sl] = jnp.maximum(
                            rows[b, r, sl] + qv[b, r, sl], 0.0
                        )
                    return 0
                lax.fori_loop(0, _CH, row, 0)

                pltpu.async_copy(
                    rows.at[b], aggr_sh.at[dsts.at[ib]], ssems[b],
                    add=True,
                )

                bn2 = (b + 2) % _SLOTS
                @pl.when(j >= 1)
                def _():
                    drain(bn2)

                @pl.when(j + 3 < n_ch_c)
                def _():
                    fetch_idx(j + 3, (ib + 3) % _ISLOTS)

                @pl.when(j + 2 < n_ch_c)
                def _():
                    wait_idx((ib + 2) % _ISLOTS)
                    fetch(j + 2, bn2, (ib + 2) % _ISLOTS)
            return 0

        lax.fori_loop(0, groups_c, group, 0)
        drain((n0 - 1) % _SLOTS)
        plsc.subcore_barrier()

        obase = s * zshare
        pltpu.sync_copy(
            aggr_sh.at[pl.ds(obase, zshare)],
            out_hbm.at[c, pl.ds(obase, zshare)],
        )

    return k(p, q, src_pad, dst_pad)


def kernel(x, edge_index, edge_attr, batch,
           W_pre0, b_pre0, W_post0, b_post0, gamma0, beta0,
           W_pre1, b_pre1, W_post1, b_post1, gamma1, beta1):
    n, d = x.shape
    e = edge_index.shape[1]
    nw = _NC * _NS
    chunk = nw * _CH * _SLOTS * _ISLOTS
    e_pad = -(-e // chunk) * chunk
    per_w = e_pad // nw
    n_ch = per_w // _CH

    src_pad = jnp.pad(edge_index[0], (0, e_pad - e))
    dst_pad = jnp.pad(edge_index[1], (0, e_pad - e), constant_values=n)

    blk_n = 1000 if n % 1000 == 0 else n
    blk_e = e
    for cand in (4000, 3200, 2560, 2500, 2000, 1600):
        if e % cand == 0:
            blk_e = cand
            break
    e_out = -(-e_pad // blk_e) * blk_e

    q0 = _q(edge_attr, W_pre0[d:], blk_e, e_out)
    p0 = _pre(x, W_pre0[:d], b_pre0, blk_n)
    parts0 = _edge_stage(p0, q0, src_pad, dst_pad, n)

    q1 = _q(edge_attr, W_pre1[d:], blk_e, e_out)

    t0, st0 = _post(parts0, W_post0, b_post0, blk_n, n)
    p1 = _mid(t0, st0, gamma0, beta0, W_pre1[:d], b_pre1, blk_n)
    parts1 = _edge_stage(p1, q1, src_pad, dst_pad, n)
    t1, st1 = _post(parts1, W_post1, b_post1, blk_n, n)

    return _readout(t1, st1, gamma1, beta1, batch, blk_n)
